# Initial kernel scaffold; baseline (speedup 1.0000x reference)
import jax
import jax.numpy as jnp
from jax.experimental import pallas as pl


def kernel(x):
    raise NotImplementedError("write your pallas kernel here")


# speedup vs baseline: 3.9093x; 3.9093x over previous
import functools

import jax
import jax.numpy as jnp
from jax import lax
from jax.experimental import pallas as pl
from jax.experimental.pallas import tpu as pltpu
from jax.experimental.pallas import tpu_sc as plsc

_K0 = 0x03D7B32D
_K1 = 0xADD083F4

_LANES = 16


def _threefry_bits_u32(x1):
    ks = (
        jnp.uint32(_K0),
        jnp.uint32(_K1),
        jnp.uint32(_K0 ^ _K1 ^ 0x1BD11BDA),
    )
    rots = ((13, 15, 26, 6), (17, 29, 16, 24))
    x0 = jnp.zeros(x1.shape, jnp.uint32) + ks[0]
    x1 = x1 + ks[1]
    for g in range(5):
        for r in rots[g % 2]:
            x0 = x0 + x1
            x1 = (x1 << jnp.uint32(r)) | (x1 >> jnp.uint32(32 - r))
            x1 = x1 ^ x0
        x0 = x0 + ks[(g + 1) % 3]
        x1 = x1 + ks[(g + 2) % 3] + jnp.uint32(g + 1)
    return x0 ^ x1


def _make_sc_kernel(b_dim, c_dim, h_dim, w_dim):
    hp = h_dim // 2
    wp = w_dim // 2
    l_dim = hp * wp
    n_jv = wp // _LANES
    tasks = b_dim * hp
    mesh = plsc.VectorSubcoreMesh(core_axis_name="c", subcore_axis_name="s")
    n_workers = mesh.num_cores * mesh.num_subcores
    tpw = tasks // n_workers
    assert tpw * n_workers == tasks and n_jv * _LANES == wp

    def body(x_hbm, o_hbm, xbuf, obuf):
        cid = lax.axis_index("c")
        sid = lax.axis_index("s")
        wid = sid * mesh.num_cores + cid
        lane = lax.iota(jnp.int32, _LANES)

        def task_body(t, carry):
            b = t // hp
            i = t % hp
            pltpu.sync_copy(x_hbm.at[b, :, i, :], xbuf)
            rowidx = []
            for jv in range(n_jv):
                j = jv * _LANES + lane
                f = (b * l_dim + i * wp + j).astype(jnp.uint32)
                k = (_threefry_bits_u32(f) & jnp.uint32(3)).astype(jnp.int32)
                di = k >> 1
                dj = k & 1
                rowidx.append(di * w_dim + 2 * j + dj)

            def ch_body(c, carry2):
                cs = jnp.full((_LANES,), 0, jnp.int32) + c
                for jv in range(n_jv):
                    vals = plsc.load_gather(xbuf, [cs, rowidx[jv]])
                    obuf[c, pl.ds(jv * _LANES, _LANES)] = vals
                return carry2

            lax.fori_loop(0, c_dim, ch_body, 0, unroll=False)
            pltpu.sync_copy(obuf, o_hbm.at[b, :, i, :])
            return carry

        lax.fori_loop(wid * tpw, (wid + 1) * tpw, task_body, 0, unroll=False)

    ker = pl.kernel(
        body,
        out_type=jax.ShapeDtypeStruct((b_dim, c_dim, hp, wp), jnp.float32),
        mesh=mesh,
        compiler_params=pltpu.CompilerParams(
            use_tc_tiling_on_sc=False, needs_layout_passes=False
        ),
        scratch_types=[
            pltpu.VMEM((c_dim, 2 * w_dim), jnp.float32),
            pltpu.VMEM((c_dim, wp), jnp.float32),
        ],
    )

    def run(x):
        return ker(x.reshape(b_dim, c_dim, hp, 2 * w_dim))

    return run


@jax.jit
def kernel(x):
    b_dim, c_dim, h_dim, w_dim = x.shape
    return _make_sc_kernel(b_dim, c_dim, h_dim, w_dim)(x)
